# baseline (speedup 1.0000x reference)
import functools

import jax
import jax.numpy as jnp
from jax import lax
from jax.experimental import pallas as pl
from jax.experimental.pallas import tpu as pltpu
from jax.experimental.pallas import tpu_sc as plsc

NC = 2
R = 48
RPW = 192
SC_B = 16
TC_B = 48
SC_ROW0 = TC_B * 384


def _accum(accs, buf):
    def group(g, accs_t):
        accs_l = list(accs_t)
        r0 = g * 3
        for dr in range(3):
            for wi in range(8):
                for j in range(8):
                    a = dr * 8 + j
                    v = buf[r0 + dr, wi, pl.ds(j * 16, 16)]
                    accs_l[a] = accs_l[a] + v + v * 0.0
        return tuple(accs_l)

    return lax.fori_loop(0, R // 3, group, accs)


def _sc_pool_body(x3, out, bufa, bufb, accv, sema, semb):
    wid = lax.axis_index("s") * NC + lax.axis_index("c")
    b0 = SC_ROW0 + wid * RPW

    def startA(idx):
        pltpu.make_async_copy(x3.at[pl.ds(idx, R)], bufa, sema).start()

    def startB(idx):
        pltpu.make_async_copy(x3.at[pl.ds(idx, R)], bufb, semb).start()

    def waitA():
        pltpu.make_async_copy(x3.at[pl.ds(b0, R)], bufa, sema).wait()

    def waitB():
        pltpu.make_async_copy(x3.at[pl.ds(b0, R)], bufb, semb).wait()

    startA(b0)
    startB(b0 + R)
    zero = jnp.zeros((16,), jnp.float32)
    accs = (zero,) * 24
    waitA()
    accs = _accum(accs, bufa)
    startA(b0 + 2 * R)
    waitB()
    accs = _accum(accs, bufb)
    startB(b0 + 3 * R)
    waitA()
    accs = _accum(accs, bufa)
    waitB()
    accs = _accum(accs, bufb)
    for a in range(24):
        ct, j = a // 8, a % 8
        accv[pl.ds(ct * 128 + j * 16, 16)] = accs[a]
    pltpu.sync_copy(accv, out.at[(wid % 2) * 16 + wid // 2])


def _sc_pool(x3):
    mesh = plsc.VectorSubcoreMesh(core_axis_name="c", subcore_axis_name="s")
    f = functools.partial(
        pl.kernel,
        mesh=mesh,
        out_type=jax.ShapeDtypeStruct((32, 384), jnp.float32),
        scratch_types=[
            pltpu.VMEM((R, 8, 128), jnp.float32),
            pltpu.VMEM((R, 8, 128), jnp.float32),
            pltpu.VMEM((384,), jnp.float32),
            pltpu.SemaphoreType.DMA,
            pltpu.SemaphoreType.DMA,
        ],
    )(_sc_pool_body)
    return f(x3)


def _tc_body(x_ref, w_ref, o_ref):
    inv = 1.0 / x_ref.shape[1]
    s = jnp.sum(x_ref[...], axis=1)
    o_ref[...] = jax.lax.dot_general(
        s, w_ref[...],
        dimension_numbers=(((1,), (1,)), ((), ())),
        preferred_element_type=jnp.float32,
    ) * inv


def _tc_epilogue(t_ref, p_ref, w_ref, o_ref):
    o_ref[0:TC_B, :] = t_ref[0:TC_B, :]
    s = p_ref[0] + p_ref[1]
    o_ref[TC_B:, :] = jax.lax.dot_general(
        s, w_ref[...],
        dimension_numbers=(((1,), (1,)), ((), ())),
        preferred_element_type=jnp.float32,
    ) * (1.0 / 1024.0)


def kernel(x, W):
    B, C, H, Wsp = x.shape
    S = H * Wsp
    E = W.shape[0]
    xv = jnp.transpose(x, (0, 2, 3, 1)).reshape(B, S, C)
    x3 = jnp.transpose(
        xv.reshape(B, H, Wsp // 8, 8, C // 128, 128), (0, 1, 2, 4, 3, 5)
    ).reshape(B * H * (Wsp // 8) * (C // 128), 8, 128)
    partial = _sc_pool(x3)
    Bblk = 8
    tc48 = pl.pallas_call(
        _tc_body,
        grid=(B // Bblk,),
        in_specs=[
            pl.BlockSpec((Bblk, S, C), lambda i: (i, 0, 0)),
            pl.BlockSpec((E, C), lambda i: (0, 0)),
        ],
        out_specs=pl.BlockSpec((Bblk, E), lambda i: (i, 0)),
        out_shape=jax.ShapeDtypeStruct((B, E), jnp.float32),
    )(xv, W)
    p3 = partial.reshape(2, 16, C)
    return pl.pallas_call(
        _tc_epilogue,
        grid=(1,),
        in_specs=[
            pl.BlockSpec((B, E), lambda i: (0, 0)),
            pl.BlockSpec((2, 16, C), lambda i: (0, 0, 0)),
            pl.BlockSpec((E, C), lambda i: (0, 0)),
        ],
        out_specs=pl.BlockSpec((B, E), lambda i: (0, 0)),
        out_shape=jax.ShapeDtypeStruct((B, E), jnp.float32),
    )(tc48, p3, W)


# speedup vs baseline: 1.7774x; 1.7774x over previous
import jax
import jax.numpy as jnp
from jax.experimental import pallas as pl


def _tc_body(x_ref, w_ref, o_ref):
    inv = 1.0 / x_ref.shape[1]
    s = jnp.sum(x_ref[...], axis=1)
    o_ref[...] = jax.lax.dot_general(
        s, w_ref[...],
        dimension_numbers=(((1,), (1,)), ((), ())),
        preferred_element_type=jnp.float32,
    ) * inv


def kernel(x, W):
    B, C, H, Wsp = x.shape
    S = H * Wsp
    E = W.shape[0]
    xv = jnp.transpose(x, (0, 2, 3, 1)).reshape(B, S, C)
    Bblk = 8
    return pl.pallas_call(
        _tc_body,
        grid=(B // Bblk,),
        in_specs=[
            pl.BlockSpec((Bblk, S, C), lambda i: (i, 0, 0)),
            pl.BlockSpec((E, C), lambda i: (0, 0)),
        ],
        out_specs=pl.BlockSpec((Bblk, E), lambda i: (i, 0)),
        out_shape=jax.ShapeDtypeStruct((B, E), jnp.float32),
    )(xv, W)
